# baseline (speedup 1.0000x reference)
import jax
import jax.numpy as jnp
from jax import lax
from jax.experimental import pallas as pl
from jax.experimental.pallas import tpu as pltpu
from jax.experimental.pallas import tpu_sc as plsc

NC = 2
NS = 16

N_NODES = 10000
N_EDGES = 320000
NPAD = 10240
NOWN = NPAD // NC
ACCR = 5248
CH = 128
NCHUNK = 158
EPT = NCHUNK * CH
E_PAD = NS * EPT
OUTR = NOWN // NS
ZCH = 8


def _edge_scatter(D, gather):
    mesh = plsc.VectorSubcoreMesh(
        core_axis_name="c", subcore_axis_name="s", num_cores=NC, num_subcores=NS
    )

    scratch = [
        pltpu.VMEM((NCHUNK, CH), jnp.int32),
        pltpu.VMEM((ZCH if gather else CH, D), jnp.float32),
        pltpu.VMEM_SHARED((ACCR, D), jnp.float32),
    ]
    if gather:
        scratch = [
            pltpu.VMEM((NCHUNK, CH), jnp.int32),
            pltpu.VMEM((2, CH, D), jnp.float32),
            pltpu.SemaphoreType.DMA,
            pltpu.SemaphoreType.DMA,
        ] + scratch

    def body(*refs):
        if gather:
            (g_hbm, src_hbm, dst_hbm, out_hbm, srcv, rows,
             gsem0, gsem1, dstv, zbuf, acc) = refs
        else:
            (dst_hbm, out_hbm, dstv, zbuf, acc) = refs

        cid = lax.axis_index("c")
        sid = lax.axis_index("s")

        pltpu.sync_copy(dst_hbm.at[sid], dstv)
        if gather:
            pltpu.sync_copy(src_hbm.at[sid], srcv)

        base = cid * NOWN
        lane = lax.iota(jnp.int32, 16)
        dump = NOWN + (lane & 7)

        def remap(r, _):
            for c in range(CH // 16):
                v = dstv[r, pl.ds(c * 16, 16)] - base
                ok = (v >= 0) & (v < NOWN)
                dstv[r, pl.ds(c * 16, 16)] = jnp.where(ok, v, dump)
            return 0

        lax.fori_loop(0, NCHUNK, remap, 0)

        for r in range(ZCH):
            for c in range(D // 16):
                zbuf[r, pl.ds(c * 16, 16)] = jnp.zeros((16,), jnp.float32)
        zpt = ACCR // NS
        for j in range(zpt // ZCH):
            pltpu.sync_copy(
                zbuf.at[pl.ds(0, ZCH)],
                acc.at[pl.ds(sid * zpt + j * ZCH, ZCH)],
            )
        plsc.subcore_barrier()

        if gather:
            sems = (gsem0, gsem1)

            def issue(ch, b):
                pltpu.async_copy(g_hbm.at[srcv.at[ch]], rows.at[b], sems[b])

            def wait(ch, b):
                pltpu.make_async_copy(
                    g_hbm.at[srcv.at[ch]], rows.at[b], sems[b]
                ).wait()

            issue(0, 0)
            issue(1, 1)

            def loop(chp, _):
                for b in (0, 1):
                    ch = 2 * chp + b

                    @pl.when(chp < NCHUNK // 2 - 1)
                    def _():
                        issue(ch + 2, b)

                    wait(ch, b)
                    pltpu.sync_copy(rows.at[b], acc.at[dstv.at[ch]], add=True)
                return 0

            lax.fori_loop(0, NCHUNK // 2, loop, 0)
        else:
            for r in range(CH):
                zbuf[r, pl.ds(0, 16)] = jnp.ones((16,), jnp.float32)

            def loop(ch, _):
                pltpu.sync_copy(zbuf, acc.at[dstv.at[ch]], add=True)
                return 0

            lax.fori_loop(0, NCHUNK, loop, 0)

        plsc.subcore_barrier()
        pltpu.sync_copy(
            acc.at[pl.ds(sid * OUTR, OUTR)],
            out_hbm.at[cid, pl.ds(sid * OUTR, OUTR)],
        )

    out = jax.ShapeDtypeStruct((NC, NOWN, D), jnp.float32)
    return pl.kernel(body, out_type=out, mesh=mesh, scratch_types=scratch,
                     name=f"gcn_edge_scatter_d{D}" if gather else "gcn_degree")


R = 640
GRID = NPAD // R


def _pmap(i):
    return (i // 8, i % 8, 0)


def _tc_first(degp, x, W1):
    def body(deg_ref, x_ref, w_ref, g_ref, dis_ref):
        deg = deg_ref[0, :, 0:1] + 1.0
        dis = lax.rsqrt(deg)
        dis_ref[...] = dis
        g = dis * jnp.dot(
            x_ref[...], w_ref[...], preferred_element_type=jnp.float32
        )
        g_ref[...] = jnp.concatenate(
            [g, jnp.zeros((R, 64), jnp.float32)], axis=1
        )

    return pl.pallas_call(
        body,
        grid=(GRID,),
        in_specs=[
            pl.BlockSpec((1, R, 16), _pmap),
            pl.BlockSpec((R, 128), lambda i: (i, 0)),
            pl.BlockSpec((128, 64), lambda i: (0, 0)),
        ],
        out_specs=[
            pl.BlockSpec((R, 128), lambda i: (i, 0)),
            pl.BlockSpec((R, 1), lambda i: (i, 0)),
        ],
        out_shape=[
            jax.ShapeDtypeStruct((NPAD, 128), jnp.float32),
            jax.ShapeDtypeStruct((NPAD, 1), jnp.float32),
        ],
        name="gcn_tc_first",
    )(degp, x, W1)


def _tc_mid(p, g, dis, b, W, Din, Dout):
    def body(p_ref, g_ref, dis_ref, b_ref, w_ref, o_ref):
        dis = dis_ref[...]
        s = (p_ref[0] + g_ref[...])[:, :Din]
        h = jnp.maximum(dis * s + b_ref[...], 0.0)
        o = dis * jnp.dot(h, w_ref[...], preferred_element_type=jnp.float32)
        if Dout < 128:
            o = jnp.concatenate(
                [o, jnp.zeros((R, 128 - Dout), jnp.float32)], axis=1
            )
        o_ref[...] = o

    return pl.pallas_call(
        body,
        grid=(GRID,),
        in_specs=[
            pl.BlockSpec((1, R, 128), _pmap),
            pl.BlockSpec((R, 128), lambda i: (i, 0)),
            pl.BlockSpec((R, 1), lambda i: (i, 0)),
            pl.BlockSpec((1, Din), lambda i: (0, 0)),
            pl.BlockSpec((Din, Dout), lambda i: (0, 0)),
        ],
        out_specs=pl.BlockSpec((R, 128), lambda i: (i, 0)),
        out_shape=jax.ShapeDtypeStruct((NPAD, 128), jnp.float32),
        name=f"gcn_tc_mid_{Din}_{Dout}",
    )(p, g, dis, b, W)


def _tc_final(p, g, dis, b4, gf, fc1_W, fc1_b, fc2_W, fc2_b):
    def body(p_ref, g_ref, dis_ref, b_ref, gf_ref, w1_ref, b1_ref,
             w2_ref, b2_ref, o_ref, acc_ref):
        i = pl.program_id(0)
        dis = dis_ref[...]
        s = (p_ref[0] + g_ref[...])[:, :64]
        h = jnp.maximum(dis * s + b_ref[...], 0.0)
        rowid = i * R + lax.broadcasted_iota(jnp.int32, (R, 1), 0)
        h = jnp.where(rowid < N_NODES, h, 0.0)
        part = jnp.sum(h, axis=0, keepdims=True)

        @pl.when(i == 0)
        def _():
            acc_ref[...] = jnp.zeros_like(acc_ref)

        acc_ref[...] += part

        @pl.when(i == GRID - 1)
        def _():
            pooled = acc_ref[...] / float(N_NODES)
            v = jnp.concatenate([pooled, gf_ref[...]], axis=1)
            v8 = jnp.broadcast_to(v, (8, 80))
            z = jnp.dot(v8, w1_ref[...], preferred_element_type=jnp.float32)
            z = jnp.maximum(z[0:1] + b1_ref[...], 0.0)
            z8 = jnp.broadcast_to(z, (8, 128))
            z2 = jnp.dot(z8, w2_ref[...], preferred_element_type=jnp.float32)
            o_ref[...] = z2[0:1] + b2_ref[...]

    return pl.pallas_call(
        body,
        grid=(GRID,),
        in_specs=[
            pl.BlockSpec((1, R, 128), _pmap),
            pl.BlockSpec((R, 128), lambda i: (i, 0)),
            pl.BlockSpec((R, 1), lambda i: (i, 0)),
            pl.BlockSpec((1, 64), lambda i: (0, 0)),
            pl.BlockSpec((1, 16), lambda i: (0, 0)),
            pl.BlockSpec((80, 128), lambda i: (0, 0)),
            pl.BlockSpec((1, 128), lambda i: (0, 0)),
            pl.BlockSpec((128, 128), lambda i: (0, 0)),
            pl.BlockSpec((1, 128), lambda i: (0, 0)),
        ],
        out_specs=pl.BlockSpec((1, 128), lambda i: (0, 0)),
        out_shape=jax.ShapeDtypeStruct((1, 128), jnp.float32),
        scratch_shapes=[pltpu.VMEM((1, 64), jnp.float32)],
        name="gcn_tc_final",
    )(p, g, dis, b4, gf, fc1_W, fc1_b, fc2_W, fc2_b)


_deg_kernel = _edge_scatter(16, gather=False)
_scatter128 = _edge_scatter(128, gather=True)


@jax.jit
def kernel(x, edge_index, global_features, W1, b1, W2, b2, W3, b3, W4, b4,
           fc1_W, fc1_b, fc2_W, fc2_b):
    npadE = E_PAD - N_EDGES
    src = jnp.concatenate(
        [edge_index[0].astype(jnp.int32), jnp.zeros((npadE,), jnp.int32)]
    ).reshape(NS, NCHUNK, CH)
    dst = jnp.concatenate(
        [edge_index[1].astype(jnp.int32),
         jnp.full((npadE,), 2 * NOWN, jnp.int32)]
    ).reshape(NS, NCHUNK, CH)
    xp = jnp.concatenate(
        [x, jnp.zeros((NPAD - N_NODES, x.shape[1]), x.dtype)], axis=0
    )

    degp = _deg_kernel(dst)
    g1, dis = _tc_first(degp, xp, W1)

    p1 = _scatter128(g1, src, dst)
    g2 = _tc_mid(p1, g1, dis, b1.reshape(1, -1), W2, 64, 128)
    p2 = _scatter128(g2, src, dst)
    g3 = _tc_mid(p2, g2, dis, b2.reshape(1, -1), W3, 128, 128)
    p3 = _scatter128(g3, src, dst)
    g4 = _tc_mid(p3, g3, dis, b3.reshape(1, -1), W4, 128, 64)
    p4 = _scatter128(g4, src, dst)

    gf = global_features.reshape(1, -1)
    return _tc_final(p4, g4, dis, b4.reshape(1, -1), gf,
                     fc1_W, fc1_b.reshape(1, -1), fc2_W, fc2_b.reshape(1, -1))


# speedup vs baseline: 2.2184x; 2.2184x over previous
import jax
import jax.numpy as jnp
from jax import lax
from jax.experimental import pallas as pl
from jax.experimental.pallas import tpu as pltpu
from jax.experimental.pallas import tpu_sc as plsc

NC = 2
NS = 16

N_NODES = 10000
N_EDGES = 320000
NPAD = 10240
NOWN = NPAD // NC
ACCR = 5248
EPT = N_EDGES // NS
CH = 80
NCHUNK = EPT // CH
OUTR = NOWN // NS
ZCH = 8


def _edge_scatter(D, gather):
    mesh = plsc.VectorSubcoreMesh(
        core_axis_name="c", subcore_axis_name="s", num_cores=NC, num_subcores=NS
    )

    scratch = [
        pltpu.VMEM((NCHUNK, CH), jnp.int32),
        pltpu.VMEM((ZCH if gather else CH, D), jnp.float32),
        pltpu.VMEM_SHARED((ACCR, D), jnp.float32),
    ]
    if gather:
        scratch = [
            pltpu.VMEM((NCHUNK, CH), jnp.int32),
            pltpu.VMEM((2, CH, D), jnp.float32),
            pltpu.SemaphoreType.DMA,
            pltpu.SemaphoreType.DMA,
        ] + scratch

    def body(*refs):
        if gather:
            (g_hbm, src_hbm, dst_hbm, out_hbm, srcv, rows,
             gsem0, gsem1, dstv, zbuf, acc) = refs
        else:
            (dst_hbm, out_hbm, dstv, zbuf, acc) = refs

        cid = lax.axis_index("c")
        sid = lax.axis_index("s")

        pltpu.sync_copy(dst_hbm.at[sid], dstv)
        if gather:
            pltpu.sync_copy(src_hbm.at[sid], srcv)

        base = cid * NOWN
        lane = lax.iota(jnp.int32, 16)
        dump = NOWN + (lane & 7)

        def remap(r, _):
            for c in range(CH // 16):
                v = dstv[r, pl.ds(c * 16, 16)] - base
                ok = (v >= 0) & (v < NOWN)
                dstv[r, pl.ds(c * 16, 16)] = jnp.where(ok, v, dump)
            return 0

        lax.fori_loop(0, NCHUNK, remap, 0)

        for r in range(ZCH):
            for c in range(D // 16):
                zbuf[r, pl.ds(c * 16, 16)] = jnp.zeros((16,), jnp.float32)
        zpt = ACCR // NS
        for j in range(zpt // ZCH):
            pltpu.sync_copy(
                zbuf.at[pl.ds(0, ZCH)],
                acc.at[pl.ds(sid * zpt + j * ZCH, ZCH)],
            )
        plsc.subcore_barrier()

        if gather:
            sems = (gsem0, gsem1)

            def issue(ch, b):
                pltpu.async_copy(g_hbm.at[srcv.at[ch]], rows.at[b], sems[b])

            def wait(ch, b):
                pltpu.make_async_copy(
                    g_hbm.at[srcv.at[ch]], rows.at[b], sems[b]
                ).wait()

            issue(0, 0)
            issue(1, 1)

            def loop(chp, _):
                for b in (0, 1):
                    ch = 2 * chp + b

                    @pl.when(chp < NCHUNK // 2 - 1)
                    def _():
                        issue(ch + 2, b)

                    wait(ch, b)
                    pltpu.sync_copy(rows.at[b], acc.at[dstv.at[ch]], add=True)
                return 0

            lax.fori_loop(0, NCHUNK // 2, loop, 0)
        else:
            for r in range(CH):
                zbuf[r, pl.ds(0, 16)] = jnp.ones((16,), jnp.float32)

            def loop(ch, _):
                pltpu.sync_copy(zbuf, acc.at[dstv.at[ch]], add=True)
                return 0

            lax.fori_loop(0, NCHUNK, loop, 0)

        plsc.subcore_barrier()
        pltpu.sync_copy(
            acc.at[pl.ds(sid * OUTR, OUTR)],
            out_hbm.at[cid, pl.ds(sid * OUTR, OUTR)],
        )

    out = jax.ShapeDtypeStruct((NC, NOWN, D), jnp.float32)
    return pl.kernel(body, out_type=out, mesh=mesh, scratch_types=scratch,
                     name=f"gcn_edge_scatter_d{D}" if gather else "gcn_degree")


R = 640
GRID = NPAD // R


def _pmap(i):
    return (i // 8, i % 8, 0)


def _tc_first(degp, x, W1):
    def body(deg_ref, x_ref, w_ref, g_ref, dis_ref):
        deg = deg_ref[0, :, 0:1] + 1.0
        dis = lax.rsqrt(deg)
        dis_ref[...] = dis
        g = dis * jnp.dot(
            x_ref[...], w_ref[...], preferred_element_type=jnp.float32
        )
        g_ref[...] = jnp.concatenate(
            [g, jnp.zeros((R, 64), jnp.float32)], axis=1
        )

    return pl.pallas_call(
        body,
        grid=(GRID,),
        in_specs=[
            pl.BlockSpec((1, R, 16), _pmap),
            pl.BlockSpec((R, 128), lambda i: (i, 0)),
            pl.BlockSpec((128, 64), lambda i: (0, 0)),
        ],
        out_specs=[
            pl.BlockSpec((R, 128), lambda i: (i, 0)),
            pl.BlockSpec((R, 1), lambda i: (i, 0)),
        ],
        out_shape=[
            jax.ShapeDtypeStruct((NPAD, 128), jnp.float32),
            jax.ShapeDtypeStruct((NPAD, 1), jnp.float32),
        ],
        name="gcn_tc_first",
    )(degp, x, W1)


def _tc_mid(p, g, dis, b, W, Din, Dout):
    def body(p_ref, g_ref, dis_ref, b_ref, w_ref, o_ref):
        dis = dis_ref[...]
        s = (p_ref[0] + g_ref[...])[:, :Din]
        h = jnp.maximum(dis * s + b_ref[...], 0.0)
        o = dis * jnp.dot(h, w_ref[...], preferred_element_type=jnp.float32)
        if Dout < 128:
            o = jnp.concatenate(
                [o, jnp.zeros((R, 128 - Dout), jnp.float32)], axis=1
            )
        o_ref[...] = o

    return pl.pallas_call(
        body,
        grid=(GRID,),
        in_specs=[
            pl.BlockSpec((1, R, 128), _pmap),
            pl.BlockSpec((R, 128), lambda i: (i, 0)),
            pl.BlockSpec((R, 1), lambda i: (i, 0)),
            pl.BlockSpec((1, Din), lambda i: (0, 0)),
            pl.BlockSpec((Din, Dout), lambda i: (0, 0)),
        ],
        out_specs=pl.BlockSpec((R, 128), lambda i: (i, 0)),
        out_shape=jax.ShapeDtypeStruct((NPAD, 128), jnp.float32),
        name=f"gcn_tc_mid_{Din}_{Dout}",
    )(p, g, dis, b, W)


def _tc_final(p, g, dis, b4, gf, fc1_W, fc1_b, fc2_W, fc2_b):
    def body(p_ref, g_ref, dis_ref, b_ref, gf_ref, w1_ref, b1_ref,
             w2_ref, b2_ref, o_ref, acc_ref):
        i = pl.program_id(0)
        dis = dis_ref[...]
        s = (p_ref[0] + g_ref[...])[:, :64]
        h = jnp.maximum(dis * s + b_ref[...], 0.0)
        rowid = i * R + lax.broadcasted_iota(jnp.int32, (R, 1), 0)
        h = jnp.where(rowid < N_NODES, h, 0.0)
        part = jnp.sum(h, axis=0, keepdims=True)

        @pl.when(i == 0)
        def _():
            acc_ref[...] = jnp.zeros_like(acc_ref)

        acc_ref[...] += part

        @pl.when(i == GRID - 1)
        def _():
            pooled = acc_ref[...] / float(N_NODES)
            v = jnp.concatenate([pooled, gf_ref[...]], axis=1)
            v8 = jnp.broadcast_to(v, (8, 80))
            z = jnp.dot(v8, w1_ref[...], preferred_element_type=jnp.float32)
            z = jnp.maximum(z[0:1] + b1_ref[...], 0.0)
            z8 = jnp.broadcast_to(z, (8, 128))
            z2 = jnp.dot(z8, w2_ref[...], preferred_element_type=jnp.float32)
            o_ref[...] = z2[0:1] + b2_ref[...]

    return pl.pallas_call(
        body,
        grid=(GRID,),
        in_specs=[
            pl.BlockSpec((1, R, 128), _pmap),
            pl.BlockSpec((R, 128), lambda i: (i, 0)),
            pl.BlockSpec((R, 1), lambda i: (i, 0)),
            pl.BlockSpec((1, 64), lambda i: (0, 0)),
            pl.BlockSpec((1, 16), lambda i: (0, 0)),
            pl.BlockSpec((80, 128), lambda i: (0, 0)),
            pl.BlockSpec((1, 128), lambda i: (0, 0)),
            pl.BlockSpec((128, 128), lambda i: (0, 0)),
            pl.BlockSpec((1, 128), lambda i: (0, 0)),
        ],
        out_specs=pl.BlockSpec((1, 128), lambda i: (0, 0)),
        out_shape=jax.ShapeDtypeStruct((1, 128), jnp.float32),
        scratch_shapes=[pltpu.VMEM((1, 64), jnp.float32)],
        name="gcn_tc_final",
    )(p, g, dis, b4, gf, fc1_W, fc1_b, fc2_W, fc2_b)


_deg_kernel = _edge_scatter(16, gather=False)
_scatter128 = _edge_scatter(128, gather=True)


@jax.jit
def kernel(x, edge_index, global_features, W1, b1, W2, b2, W3, b3, W4, b4,
           fc1_W, fc1_b, fc2_W, fc2_b):
    src = edge_index[0].astype(jnp.int32).reshape(NS, NCHUNK, CH)
    dst = edge_index[1].astype(jnp.int32).reshape(NS, NCHUNK, CH)
    xp = jnp.concatenate(
        [x, jnp.zeros((NPAD - N_NODES, x.shape[1]), x.dtype)], axis=0
    )

    degp = _deg_kernel(dst)
    g1, dis = _tc_first(degp, xp, W1)

    p1 = _scatter128(g1, src, dst)
    g2 = _tc_mid(p1, g1, dis, b1.reshape(1, -1), W2, 64, 128)
    p2 = _scatter128(g2, src, dst)
    g3 = _tc_mid(p2, g2, dis, b2.reshape(1, -1), W3, 128, 128)
    p3 = _scatter128(g3, src, dst)
    g4 = _tc_mid(p3, g3, dis, b3.reshape(1, -1), W4, 128, 64)
    p4 = _scatter128(g4, src, dst)

    gf = global_features.reshape(1, -1)
    return _tc_final(p4, g4, dis, b4.reshape(1, -1), gf,
                     fc1_W, fc1_b.reshape(1, -1), fc2_W, fc2_b.reshape(1, -1))


# speedup vs baseline: 2.5330x; 1.1418x over previous
import jax
import jax.numpy as jnp
from jax import lax
from jax.experimental import pallas as pl
from jax.experimental.pallas import tpu as pltpu
from jax.experimental.pallas import tpu_sc as plsc

NC = 2
NS = 16

N_NODES = 10000
N_EDGES = 320000
NPAD = 10240
NOWN = NPAD // NC
ACCR = 5248
EPT = N_EDGES // NS
CH = 80
NCHUNK = EPT // CH
OUTR = NOWN // NS
ZCH = 8


def _edge_scatter(D, gather, tc_tiling=True):
    mesh = plsc.VectorSubcoreMesh(
        core_axis_name="c", subcore_axis_name="s", num_cores=NC, num_subcores=NS
    )

    scratch = [
        pltpu.VMEM((NCHUNK, CH), jnp.int32),
        pltpu.VMEM((ZCH if gather else CH, D), jnp.float32),
        pltpu.VMEM_SHARED((ACCR, D), jnp.float32),
    ]
    if gather:
        scratch = [
            pltpu.VMEM((NCHUNK, CH), jnp.int32),
            pltpu.VMEM((2, CH, D), jnp.float32),
            pltpu.SemaphoreType.DMA,
            pltpu.SemaphoreType.DMA,
        ] + scratch

    def body(*refs):
        if gather:
            (g_hbm, src_hbm, dst_hbm, out_hbm, srcv, rows,
             gsem0, gsem1, dstv, zbuf, acc) = refs
        else:
            (dst_hbm, out_hbm, dstv, zbuf, acc) = refs

        cid = lax.axis_index("c")
        sid = lax.axis_index("s")

        pltpu.sync_copy(dst_hbm.at[sid], dstv)
        if gather:
            pltpu.sync_copy(src_hbm.at[sid], srcv)

        base = cid * NOWN
        lane = lax.iota(jnp.int32, 16)
        dump = NOWN + (lane & 7)

        def remap(r, _):
            for c in range(CH // 16):
                v = dstv[r, pl.ds(c * 16, 16)] - base
                ok = (v >= 0) & (v < NOWN)
                dstv[r, pl.ds(c * 16, 16)] = jnp.where(ok, v, dump)
            return 0

        lax.fori_loop(0, NCHUNK, remap, 0)

        for r in range(ZCH):
            for c in range(D // 16):
                zbuf[r, pl.ds(c * 16, 16)] = jnp.zeros((16,), jnp.float32)
        zpt = ACCR // NS
        for j in range(zpt // ZCH):
            pltpu.sync_copy(
                zbuf.at[pl.ds(0, ZCH)],
                acc.at[pl.ds(sid * zpt + j * ZCH, ZCH)],
            )
        plsc.subcore_barrier()

        if gather:
            sems = (gsem0, gsem1)

            def issue(ch, b):
                pltpu.async_copy(g_hbm.at[srcv.at[ch]], rows.at[b], sems[b])

            def wait(ch, b):
                pltpu.make_async_copy(
                    g_hbm.at[srcv.at[ch]], rows.at[b], sems[b]
                ).wait()

            issue(0, 0)
            issue(1, 1)

            def loop(chp, _):
                for b in (0, 1):
                    ch = 2 * chp + b

                    @pl.when(chp < NCHUNK // 2 - 1)
                    def _():
                        issue(ch + 2, b)

                    wait(ch, b)
                    pltpu.sync_copy(rows.at[b], acc.at[dstv.at[ch]], add=True)
                return 0

            lax.fori_loop(0, NCHUNK // 2, loop, 0)
        else:
            for r in range(CH):
                zbuf[r, pl.ds(0, 16)] = jnp.ones((16,), jnp.float32)

            def loop(ch, _):
                pltpu.sync_copy(zbuf, acc.at[dstv.at[ch]], add=True)
                return 0

            lax.fori_loop(0, NCHUNK, loop, 0)

        plsc.subcore_barrier()
        pltpu.sync_copy(
            acc.at[pl.ds(sid * OUTR, OUTR)],
            out_hbm.at[cid, pl.ds(sid * OUTR, OUTR)],
        )

    out = jax.ShapeDtypeStruct((NC, NOWN, D), jnp.float32)
    params = pltpu.CompilerParams(use_tc_tiling_on_sc=tc_tiling)
    return pl.kernel(body, out_type=out, mesh=mesh, scratch_types=scratch,
                     compiler_params=params,
                     name=f"gcn_edge_scatter_d{D}" if gather else "gcn_degree")


R = 640
GRID = NPAD // R


def _pmap(i):
    return (i // 8, i % 8, 0)


def _tc_first(degp, x, W1):
    def body(deg_ref, x_ref, w_ref, g_ref, dis_ref):
        deg = deg_ref[0, :, 0:1] + 1.0
        dis = lax.rsqrt(deg)
        dis_ref[...] = dis
        g_ref[...] = dis * jnp.dot(
            x_ref[...], w_ref[...], preferred_element_type=jnp.float32
        )

    return pl.pallas_call(
        body,
        grid=(GRID,),
        in_specs=[
            pl.BlockSpec((1, R, 16), _pmap),
            pl.BlockSpec((R, 128), lambda i: (i, 0)),
            pl.BlockSpec((128, 64), lambda i: (0, 0)),
        ],
        out_specs=[
            pl.BlockSpec((R, 64), lambda i: (i, 0)),
            pl.BlockSpec((R, 1), lambda i: (i, 0)),
        ],
        out_shape=[
            jax.ShapeDtypeStruct((NPAD, 64), jnp.float32),
            jax.ShapeDtypeStruct((NPAD, 1), jnp.float32),
        ],
        name="gcn_tc_first",
    )(degp, x, W1)


def _tc_mid(p, g, dis, b, W, Din, Dout):
    def body(p_ref, g_ref, dis_ref, b_ref, w_ref, o_ref):
        dis = dis_ref[...]
        h = jnp.maximum(dis * (p_ref[0] + g_ref[...]) + b_ref[...], 0.0)
        o_ref[...] = dis * jnp.dot(
            h, w_ref[...], preferred_element_type=jnp.float32
        )

    return pl.pallas_call(
        body,
        grid=(GRID,),
        in_specs=[
            pl.BlockSpec((1, R, Din), _pmap),
            pl.BlockSpec((R, Din), lambda i: (i, 0)),
            pl.BlockSpec((R, 1), lambda i: (i, 0)),
            pl.BlockSpec((1, Din), lambda i: (0, 0)),
            pl.BlockSpec((Din, Dout), lambda i: (0, 0)),
        ],
        out_specs=pl.BlockSpec((R, Dout), lambda i: (i, 0)),
        out_shape=jax.ShapeDtypeStruct((NPAD, Dout), jnp.float32),
        name=f"gcn_tc_mid_{Din}_{Dout}",
    )(p, g, dis, b, W)


def _tc_final(p, g, dis, b4, gf, fc1_W, fc1_b, fc2_W, fc2_b):
    def body(p_ref, g_ref, dis_ref, b_ref, gf_ref, w1_ref, b1_ref,
             w2_ref, b2_ref, o_ref, acc_ref):
        i = pl.program_id(0)
        dis = dis_ref[...]
        h = jnp.maximum(dis * (p_ref[0] + g_ref[...]) + b_ref[...], 0.0)
        rowid = i * R + lax.broadcasted_iota(jnp.int32, (R, 1), 0)
        h = jnp.where(rowid < N_NODES, h, 0.0)
        part = jnp.sum(h, axis=0, keepdims=True)

        @pl.when(i == 0)
        def _():
            acc_ref[...] = jnp.zeros_like(acc_ref)

        acc_ref[...] += part

        @pl.when(i == GRID - 1)
        def _():
            pooled = acc_ref[...] / float(N_NODES)
            v = jnp.concatenate([pooled, gf_ref[...]], axis=1)
            v8 = jnp.broadcast_to(v, (8, 80))
            z = jnp.dot(v8, w1_ref[...], preferred_element_type=jnp.float32)
            z = jnp.maximum(z[0:1] + b1_ref[...], 0.0)
            z8 = jnp.broadcast_to(z, (8, 128))
            z2 = jnp.dot(z8, w2_ref[...], preferred_element_type=jnp.float32)
            o_ref[...] = z2[0:1] + b2_ref[...]

    return pl.pallas_call(
        body,
        grid=(GRID,),
        in_specs=[
            pl.BlockSpec((1, R, 64), _pmap),
            pl.BlockSpec((R, 64), lambda i: (i, 0)),
            pl.BlockSpec((R, 1), lambda i: (i, 0)),
            pl.BlockSpec((1, 64), lambda i: (0, 0)),
            pl.BlockSpec((1, 16), lambda i: (0, 0)),
            pl.BlockSpec((80, 128), lambda i: (0, 0)),
            pl.BlockSpec((1, 128), lambda i: (0, 0)),
            pl.BlockSpec((128, 128), lambda i: (0, 0)),
            pl.BlockSpec((1, 128), lambda i: (0, 0)),
        ],
        out_specs=pl.BlockSpec((1, 128), lambda i: (0, 0)),
        out_shape=jax.ShapeDtypeStruct((1, 128), jnp.float32),
        scratch_shapes=[pltpu.VMEM((1, 64), jnp.float32)],
        name="gcn_tc_final",
    )(p, g, dis, b4, gf, fc1_W, fc1_b, fc2_W, fc2_b)


_deg_kernel = _edge_scatter(16, gather=False)
_scatter128 = _edge_scatter(128, gather=True)
_scatter64 = _edge_scatter(64, gather=True, tc_tiling=False)


@jax.jit
def kernel(x, edge_index, global_features, W1, b1, W2, b2, W3, b3, W4, b4,
           fc1_W, fc1_b, fc2_W, fc2_b):
    src = edge_index[0].astype(jnp.int32).reshape(NS, NCHUNK, CH)
    dst = edge_index[1].astype(jnp.int32).reshape(NS, NCHUNK, CH)
    xp = jnp.concatenate(
        [x, jnp.zeros((NPAD - N_NODES, x.shape[1]), x.dtype)], axis=0
    )

    degp = _deg_kernel(dst)
    g1, dis = _tc_first(degp, xp, W1)

    p1 = _scatter64(g1, src, dst)
    g2 = _tc_mid(p1, g1, dis, b1.reshape(1, -1), W2, 64, 128)
    p2 = _scatter128(g2, src, dst)
    g3 = _tc_mid(p2, g2, dis, b2.reshape(1, -1), W3, 128, 128)
    p3 = _scatter128(g3, src, dst)
    g4 = _tc_mid(p3, g3, dis, b3.reshape(1, -1), W4, 128, 64)
    p4 = _scatter64(g4, src, dst)

    gf = global_features.reshape(1, -1)
    return _tc_final(p4, g4, dis, b4.reshape(1, -1), gf,
                     fc1_W, fc1_b.reshape(1, -1), fc2_W, fc2_b.reshape(1, -1))


# speedup vs baseline: 2.5429x; 1.0039x over previous
import jax
import jax.numpy as jnp
from jax import lax
from jax.experimental import pallas as pl
from jax.experimental.pallas import tpu as pltpu
from jax.experimental.pallas import tpu_sc as plsc

NC = 2
NS = 16

N_NODES = 10000
N_EDGES = 320000
NPAD = 10240
NOWN = NPAD // NC
ACCR = 5248
EPT = N_EDGES // NS
CH = 80
NCHUNK = EPT // CH
OUTR = NOWN // NS
ZCH = 8


def _edge_scatter(D, gather, tc_tiling=True):
    mesh = plsc.VectorSubcoreMesh(
        core_axis_name="c", subcore_axis_name="s", num_cores=NC, num_subcores=NS
    )

    scratch = [
        pltpu.VMEM((NCHUNK, CH), jnp.int32),
        pltpu.VMEM((ZCH if gather else CH, D), jnp.float32),
        pltpu.VMEM_SHARED((ACCR, D), jnp.float32),
    ]
    if gather:
        scratch = [
            pltpu.VMEM((NCHUNK, CH), jnp.int32),
            pltpu.VMEM((2, CH, D), jnp.float32),
            pltpu.SemaphoreType.DMA,
            pltpu.SemaphoreType.DMA,
        ] + scratch

    def body(*refs):
        if gather:
            (g_hbm, src_hbm, dst_hbm, out_hbm, srcv, rows,
             gsem0, gsem1, dstv, zbuf, acc) = refs
        else:
            (dst_hbm, out_hbm, dstv, zbuf, acc) = refs

        cid = lax.axis_index("c")
        sid = lax.axis_index("s")

        pltpu.sync_copy(dst_hbm.at[sid], dstv)
        if gather:
            pltpu.sync_copy(src_hbm.at[sid], srcv)

        base = cid * NOWN
        lane = lax.iota(jnp.int32, 16)
        dump = NOWN + (lane & 7)

        def remap(r, _):
            for c in range(CH // 16):
                v = dstv[r, pl.ds(c * 16, 16)] - base
                ok = (v >= 0) & (v < NOWN)
                dstv[r, pl.ds(c * 16, 16)] = jnp.where(ok, v, dump)
            return 0

        lax.fori_loop(0, NCHUNK, remap, 0)

        for r in range(ZCH):
            for c in range(D // 16):
                zbuf[r, pl.ds(c * 16, 16)] = jnp.zeros((16,), jnp.float32)
        zpt = ACCR // NS
        for j in range(zpt // ZCH):
            pltpu.sync_copy(
                zbuf.at[pl.ds(0, ZCH)],
                acc.at[pl.ds(sid * zpt + j * ZCH, ZCH)],
            )
        plsc.subcore_barrier()

        if gather:
            sems = (gsem0, gsem1)

            def issue(ch, b):
                pltpu.async_copy(g_hbm.at[srcv.at[ch]], rows.at[b], sems[b])

            def wait(ch, b):
                pltpu.make_async_copy(
                    g_hbm.at[srcv.at[ch]], rows.at[b], sems[b]
                ).wait()

            issue(0, 0)
            issue(1, 1)

            def loop(chp, _):
                for b in (0, 1):
                    ch = 2 * chp + b

                    @pl.when(chp < NCHUNK // 2 - 1)
                    def _():
                        issue(ch + 2, b)

                    wait(ch, b)
                    pltpu.sync_copy(rows.at[b], acc.at[dstv.at[ch]], add=True)
                return 0

            lax.fori_loop(0, NCHUNK // 2, loop, 0)
        else:
            for r in range(CH):
                zbuf[r, pl.ds(0, 16)] = jnp.ones((16,), jnp.float32)

            def loop(ch, _):
                pltpu.sync_copy(zbuf, acc.at[dstv.at[ch]], add=True)
                return 0

            lax.fori_loop(0, NCHUNK, loop, 0)

        plsc.subcore_barrier()
        pltpu.sync_copy(
            acc.at[pl.ds(sid * OUTR, OUTR)],
            out_hbm.at[cid, pl.ds(sid * OUTR, OUTR)],
        )

    out = jax.ShapeDtypeStruct((NC, NOWN, D), jnp.float32)
    params = pltpu.CompilerParams(use_tc_tiling_on_sc=tc_tiling)
    return pl.kernel(body, out_type=out, mesh=mesh, scratch_types=scratch,
                     compiler_params=params,
                     name=f"gcn_edge_scatter_d{D}" if gather else "gcn_degree")


R = 640
GRID = NPAD // R


def _pmap(i):
    return (i // 8, i % 8, 0)


def _tc_first(degp, x, W1):
    def body(deg_ref, x_ref, w_ref, g_ref, dis_ref):
        deg = deg_ref[0, :, 0:1] + 1.0
        dis = lax.rsqrt(deg)
        dis_ref[...] = dis
        g_ref[...] = dis * jnp.dot(
            x_ref[...], w_ref[...], preferred_element_type=jnp.float32
        )

    return pl.pallas_call(
        body,
        grid=(GRID,),
        in_specs=[
            pl.BlockSpec((1, R, 16), _pmap),
            pl.BlockSpec((R, 128), lambda i: (i, 0)),
            pl.BlockSpec((128, 64), lambda i: (0, 0)),
        ],
        out_specs=[
            pl.BlockSpec((R, 64), lambda i: (i, 0)),
            pl.BlockSpec((R, 1), lambda i: (i, 0)),
        ],
        out_shape=[
            jax.ShapeDtypeStruct((NPAD, 64), jnp.float32),
            jax.ShapeDtypeStruct((NPAD, 1), jnp.float32),
        ],
        name="gcn_tc_first",
    )(degp, x, W1)


def _tc_mid(p, g, dis, b, W, Din, Dout):
    def body(p_ref, g_ref, dis_ref, b_ref, w_ref, o_ref):
        dis = dis_ref[...]
        h = jnp.maximum(dis * (p_ref[0] + g_ref[...]) + b_ref[...], 0.0)
        o_ref[...] = dis * jnp.dot(
            h, w_ref[...], preferred_element_type=jnp.float32
        )

    return pl.pallas_call(
        body,
        grid=(GRID,),
        in_specs=[
            pl.BlockSpec((1, R, Din), _pmap),
            pl.BlockSpec((R, Din), lambda i: (i, 0)),
            pl.BlockSpec((R, 1), lambda i: (i, 0)),
            pl.BlockSpec((1, Din), lambda i: (0, 0)),
            pl.BlockSpec((Din, Dout), lambda i: (0, 0)),
        ],
        out_specs=pl.BlockSpec((R, Dout), lambda i: (i, 0)),
        out_shape=jax.ShapeDtypeStruct((NPAD, Dout), jnp.float32),
        name=f"gcn_tc_mid_{Din}_{Dout}",
    )(p, g, dis, b, W)


def _tc_final(p, g, dis, b4, gf, fc1_W, fc1_b, fc2_W, fc2_b):
    def body(p_ref, g_ref, dis_ref, b_ref, gf_ref, w1_ref, b1_ref,
             w2_ref, b2_ref, o_ref, acc_ref):
        i = pl.program_id(0)
        dis = dis_ref[...]
        h = jnp.maximum(dis * (p_ref[0] + g_ref[...]) + b_ref[...], 0.0)
        rowid = i * R + lax.broadcasted_iota(jnp.int32, (R, 1), 0)
        h = jnp.where(rowid < N_NODES, h, 0.0)
        part = jnp.sum(h, axis=0, keepdims=True)

        @pl.when(i == 0)
        def _():
            acc_ref[...] = jnp.zeros_like(acc_ref)

        acc_ref[...] += part

        @pl.when(i == GRID - 1)
        def _():
            pooled = acc_ref[...] / float(N_NODES)
            v = jnp.concatenate([pooled, gf_ref[...]], axis=1)
            v8 = jnp.broadcast_to(v, (8, 80))
            z = jnp.dot(v8, w1_ref[...], preferred_element_type=jnp.float32)
            z = jnp.maximum(z[0:1] + b1_ref[...], 0.0)
            z8 = jnp.broadcast_to(z, (8, 128))
            z2 = jnp.dot(z8, w2_ref[...], preferred_element_type=jnp.float32)
            o_ref[...] = z2[0:1] + b2_ref[...]

    return pl.pallas_call(
        body,
        grid=(GRID,),
        in_specs=[
            pl.BlockSpec((1, R, 64), _pmap),
            pl.BlockSpec((R, 64), lambda i: (i, 0)),
            pl.BlockSpec((R, 1), lambda i: (i, 0)),
            pl.BlockSpec((1, 64), lambda i: (0, 0)),
            pl.BlockSpec((1, 16), lambda i: (0, 0)),
            pl.BlockSpec((80, 128), lambda i: (0, 0)),
            pl.BlockSpec((1, 128), lambda i: (0, 0)),
            pl.BlockSpec((128, 128), lambda i: (0, 0)),
            pl.BlockSpec((1, 128), lambda i: (0, 0)),
        ],
        out_specs=pl.BlockSpec((1, 128), lambda i: (0, 0)),
        out_shape=jax.ShapeDtypeStruct((1, 128), jnp.float32),
        scratch_shapes=[pltpu.VMEM((1, 64), jnp.float32)],
        name="gcn_tc_final",
    )(p, g, dis, b4, gf, fc1_W, fc1_b, fc2_W, fc2_b)


_deg_kernel = _edge_scatter(16, gather=False)
_scatter128 = _edge_scatter(128, gather=True, tc_tiling=False)
_scatter64 = _edge_scatter(64, gather=True, tc_tiling=False)


@jax.jit
def kernel(x, edge_index, global_features, W1, b1, W2, b2, W3, b3, W4, b4,
           fc1_W, fc1_b, fc2_W, fc2_b):
    src = edge_index[0].astype(jnp.int32).reshape(NS, NCHUNK, CH)
    dst = edge_index[1].astype(jnp.int32).reshape(NS, NCHUNK, CH)
    xp = jnp.concatenate(
        [x, jnp.zeros((NPAD - N_NODES, x.shape[1]), x.dtype)], axis=0
    )

    degp = _deg_kernel(dst)
    g1, dis = _tc_first(degp, xp, W1)

    p1 = _scatter64(g1, src, dst)
    g2 = _tc_mid(p1, g1, dis, b1.reshape(1, -1), W2, 64, 128)
    p2 = _scatter128(g2, src, dst)
    g3 = _tc_mid(p2, g2, dis, b2.reshape(1, -1), W3, 128, 128)
    p3 = _scatter128(g3, src, dst)
    g4 = _tc_mid(p3, g3, dis, b3.reshape(1, -1), W4, 128, 64)
    p4 = _scatter64(g4, src, dst)

    gf = global_features.reshape(1, -1)
    return _tc_final(p4, g4, dis, b4.reshape(1, -1), gf,
                     fc1_W, fc1_b.reshape(1, -1), fc2_W, fc2_b.reshape(1, -1))


# speedup vs baseline: 2.5728x; 1.0118x over previous
import jax
import jax.numpy as jnp
from jax import lax
from jax.experimental import pallas as pl
from jax.experimental.pallas import tpu as pltpu
from jax.experimental.pallas import tpu_sc as plsc

NC = 2
NS = 16

N_NODES = 10000
N_EDGES = 320000
NPAD = 10240
NOWN = NPAD // NC
ACCR = 5376
EPT = N_EDGES // NS
CH = 80
NCHUNK = EPT // CH
OUTR = NOWN // NS
ZCH = 48


def _edge_scatter(D, gather, tc_tiling=True):
    mesh = plsc.VectorSubcoreMesh(
        core_axis_name="c", subcore_axis_name="s", num_cores=NC, num_subcores=NS
    )

    scratch = [
        pltpu.VMEM((NCHUNK, CH), jnp.int32),
        pltpu.VMEM((ZCH if gather else CH, D), jnp.float32),
        pltpu.VMEM_SHARED((ACCR, D), jnp.float32),
    ]
    if gather:
        scratch = [
            pltpu.VMEM((NCHUNK, CH), jnp.int32),
            pltpu.VMEM((2, CH, D), jnp.float32),
            pltpu.SemaphoreType.DMA,
            pltpu.SemaphoreType.DMA,
        ] + scratch

    def body(*refs):
        if gather:
            (g_hbm, src_hbm, dst_hbm, out_hbm, srcv, rows,
             gsem0, gsem1, dstv, zbuf, acc) = refs
        else:
            (dst_hbm, out_hbm, dstv, zbuf, acc) = refs

        cid = lax.axis_index("c")
        sid = lax.axis_index("s")

        pltpu.sync_copy(dst_hbm.at[sid], dstv)
        if gather:
            pltpu.sync_copy(src_hbm.at[sid], srcv)

        base = cid * NOWN
        lane = lax.iota(jnp.int32, 16)
        dump = NOWN + (lane & 7)

        def remap(r, _):
            for c in range(CH // 16):
                v = dstv[r, pl.ds(c * 16, 16)] - base
                ok = (v >= 0) & (v < NOWN)
                dstv[r, pl.ds(c * 16, 16)] = jnp.where(ok, v, dump)
            return 0

        lax.fori_loop(0, NCHUNK, remap, 0)

        for r in range(ZCH):
            for c in range(D // 16):
                zbuf[r, pl.ds(c * 16, 16)] = jnp.zeros((16,), jnp.float32)
        zpt = ACCR // NS
        for j in range(zpt // ZCH):
            pltpu.sync_copy(
                zbuf.at[pl.ds(0, ZCH)],
                acc.at[pl.ds(sid * zpt + j * ZCH, ZCH)],
            )
        plsc.subcore_barrier()

        if gather:
            sems = (gsem0, gsem1)

            def issue(ch, b):
                pltpu.async_copy(g_hbm.at[srcv.at[ch]], rows.at[b], sems[b])

            def wait(ch, b):
                pltpu.make_async_copy(
                    g_hbm.at[srcv.at[ch]], rows.at[b], sems[b]
                ).wait()

            issue(0, 0)
            issue(1, 1)

            def loop(chp, _):
                for b in (0, 1):
                    ch = 2 * chp + b

                    @pl.when(chp < NCHUNK // 2 - 1)
                    def _():
                        issue(ch + 2, b)

                    wait(ch, b)
                    pltpu.sync_copy(rows.at[b], acc.at[dstv.at[ch]], add=True)
                return 0

            lax.fori_loop(0, NCHUNK // 2, loop, 0)
        else:
            for r in range(CH):
                zbuf[r, pl.ds(0, 16)] = jnp.ones((16,), jnp.float32)

            def loop(ch, _):
                pltpu.sync_copy(zbuf, acc.at[dstv.at[ch]], add=True)
                return 0

            lax.fori_loop(0, NCHUNK, loop, 0)

        plsc.subcore_barrier()
        pltpu.sync_copy(
            acc.at[pl.ds(sid * OUTR, OUTR)],
            out_hbm.at[cid, pl.ds(sid * OUTR, OUTR)],
        )

    out = jax.ShapeDtypeStruct((NC, NOWN, D), jnp.float32)
    params = pltpu.CompilerParams(use_tc_tiling_on_sc=tc_tiling)
    return pl.kernel(body, out_type=out, mesh=mesh, scratch_types=scratch,
                     compiler_params=params,
                     name=f"gcn_edge_scatter_d{D}" if gather else "gcn_degree")


R = 640
GRID = NPAD // R


def _pmap(i):
    return (i // 8, i % 8, 0)


def _tc_first(degp, x, W1):
    def body(deg_ref, x_ref, w_ref, g_ref, dis_ref):
        deg = deg_ref[0, :, 0:1] + 1.0
        dis = lax.rsqrt(deg)
        dis_ref[...] = dis
        g_ref[...] = dis * jnp.dot(
            x_ref[...], w_ref[...], preferred_element_type=jnp.float32
        )

    return pl.pallas_call(
        body,
        grid=(GRID,),
        in_specs=[
            pl.BlockSpec((1, R, 16), _pmap),
            pl.BlockSpec((R, 128), lambda i: (i, 0)),
            pl.BlockSpec((128, 64), lambda i: (0, 0)),
        ],
        out_specs=[
            pl.BlockSpec((R, 64), lambda i: (i, 0)),
            pl.BlockSpec((R, 1), lambda i: (i, 0)),
        ],
        out_shape=[
            jax.ShapeDtypeStruct((NPAD, 64), jnp.float32),
            jax.ShapeDtypeStruct((NPAD, 1), jnp.float32),
        ],
        name="gcn_tc_first",
    )(degp, x, W1)


def _tc_mid(p, g, dis, b, W, Din, Dout):
    def body(p_ref, g_ref, dis_ref, b_ref, w_ref, o_ref):
        dis = dis_ref[...]
        h = jnp.maximum(dis * (p_ref[0] + g_ref[...]) + b_ref[...], 0.0)
        o_ref[...] = dis * jnp.dot(
            h, w_ref[...], preferred_element_type=jnp.float32
        )

    return pl.pallas_call(
        body,
        grid=(GRID,),
        in_specs=[
            pl.BlockSpec((1, R, Din), _pmap),
            pl.BlockSpec((R, Din), lambda i: (i, 0)),
            pl.BlockSpec((R, 1), lambda i: (i, 0)),
            pl.BlockSpec((1, Din), lambda i: (0, 0)),
            pl.BlockSpec((Din, Dout), lambda i: (0, 0)),
        ],
        out_specs=pl.BlockSpec((R, Dout), lambda i: (i, 0)),
        out_shape=jax.ShapeDtypeStruct((NPAD, Dout), jnp.float32),
        name=f"gcn_tc_mid_{Din}_{Dout}",
    )(p, g, dis, b, W)


def _tc_final(p, g, dis, b4, gf, fc1_W, fc1_b, fc2_W, fc2_b):
    def body(p_ref, g_ref, dis_ref, b_ref, gf_ref, w1_ref, b1_ref,
             w2_ref, b2_ref, o_ref, acc_ref):
        i = pl.program_id(0)
        dis = dis_ref[...]
        h = jnp.maximum(dis * (p_ref[0] + g_ref[...]) + b_ref[...], 0.0)
        rowid = i * R + lax.broadcasted_iota(jnp.int32, (R, 1), 0)
        h = jnp.where(rowid < N_NODES, h, 0.0)
        part = jnp.sum(h, axis=0, keepdims=True)

        @pl.when(i == 0)
        def _():
            acc_ref[...] = jnp.zeros_like(acc_ref)

        acc_ref[...] += part

        @pl.when(i == GRID - 1)
        def _():
            pooled = acc_ref[...] / float(N_NODES)
            v = jnp.concatenate([pooled, gf_ref[...]], axis=1)
            v8 = jnp.broadcast_to(v, (8, 80))
            z = jnp.dot(v8, w1_ref[...], preferred_element_type=jnp.float32)
            z = jnp.maximum(z[0:1] + b1_ref[...], 0.0)
            z8 = jnp.broadcast_to(z, (8, 128))
            z2 = jnp.dot(z8, w2_ref[...], preferred_element_type=jnp.float32)
            o_ref[...] = z2[0:1] + b2_ref[...]

    return pl.pallas_call(
        body,
        grid=(GRID,),
        in_specs=[
            pl.BlockSpec((1, R, 64), _pmap),
            pl.BlockSpec((R, 64), lambda i: (i, 0)),
            pl.BlockSpec((R, 1), lambda i: (i, 0)),
            pl.BlockSpec((1, 64), lambda i: (0, 0)),
            pl.BlockSpec((1, 16), lambda i: (0, 0)),
            pl.BlockSpec((80, 128), lambda i: (0, 0)),
            pl.BlockSpec((1, 128), lambda i: (0, 0)),
            pl.BlockSpec((128, 128), lambda i: (0, 0)),
            pl.BlockSpec((1, 128), lambda i: (0, 0)),
        ],
        out_specs=pl.BlockSpec((1, 128), lambda i: (0, 0)),
        out_shape=jax.ShapeDtypeStruct((1, 128), jnp.float32),
        scratch_shapes=[pltpu.VMEM((1, 64), jnp.float32)],
        name="gcn_tc_final",
    )(p, g, dis, b4, gf, fc1_W, fc1_b, fc2_W, fc2_b)


_deg_kernel = _edge_scatter(16, gather=False)
_scatter128 = _edge_scatter(128, gather=True, tc_tiling=False)
_scatter64 = _edge_scatter(64, gather=True, tc_tiling=False)


@jax.jit
def kernel(x, edge_index, global_features, W1, b1, W2, b2, W3, b3, W4, b4,
           fc1_W, fc1_b, fc2_W, fc2_b):
    src = edge_index[0].astype(jnp.int32).reshape(NS, NCHUNK, CH)
    dst = edge_index[1].astype(jnp.int32).reshape(NS, NCHUNK, CH)
    xp = jnp.concatenate(
        [x, jnp.zeros((NPAD - N_NODES, x.shape[1]), x.dtype)], axis=0
    )

    degp = _deg_kernel(dst)
    g1, dis = _tc_first(degp, xp, W1)

    p1 = _scatter64(g1, src, dst)
    g2 = _tc_mid(p1, g1, dis, b1.reshape(1, -1), W2, 64, 128)
    p2 = _scatter128(g2, src, dst)
    g3 = _tc_mid(p2, g2, dis, b2.reshape(1, -1), W3, 128, 128)
    p3 = _scatter128(g3, src, dst)
    g4 = _tc_mid(p3, g3, dis, b3.reshape(1, -1), W4, 128, 64)
    p4 = _scatter64(g4, src, dst)

    gf = global_features.reshape(1, -1)
    return _tc_final(p4, g4, dis, b4.reshape(1, -1), gf,
                     fc1_W, fc1_b.reshape(1, -1), fc2_W, fc2_b.reshape(1, -1))
